# baseline (speedup 1.0000x reference)
import jax
import jax.numpy as jnp
from jax import lax
from jax.experimental import pallas as pl
from jax.experimental.pallas import tpu as pltpu
from jax.experimental.pallas import tpu_sc as plsc

D_MODEL = 64
PAD_W = 128
NUM_WORKERS = 32
LANE = 128
NBUF = 4
TOTAL = 4096 * 200
PER_WORKER = TOTAL // NUM_WORKERS
IDX_ROWS = PER_WORKER // LANE
VOCAB_N = 1000000
VOCAB_PAD = 1000064
FULL_BLOCKS = VOCAB_N // LANE
TAIL = VOCAB_N - FULL_BLOCKS * LANE
BLOCKS_PER_W = FULL_BLOCKS // NUM_WORKERS + 1


def _wid():
    return lax.axis_index("s") * 2 + lax.axis_index("c")


def _transpose_block(src_v, dst_v, n_tok):
    for g in range(n_tok // 16):
        rows = lax.iota(jnp.int32, 16) + 16 * g
        for d in range(D_MODEL):
            vals = src_v[d, pl.ds(16 * g, 16)]
            plsc.store_scatter(dst_v, [rows, jnp.full((16,), d, jnp.int32)], vals)


def _transpose_kernel(ttab_hbm, dense_hbm, tail_hbm, stage_v, stage64_v, pad_v):
    wid = _wid()

    def body(k, carry):
        j = wid + NUM_WORKERS * k

        @pl.when(j < FULL_BLOCKS)
        def _():
            col = pl.multiple_of(j * LANE, LANE)
            pltpu.sync_copy(ttab_hbm.at[:, pl.ds(col, LANE)], stage_v)
            _transpose_block(stage_v, pad_v, LANE)
            pltpu.sync_copy(pad_v, dense_hbm.at[pl.ds(col, LANE), :])

        return carry

    lax.fori_loop(0, BLOCKS_PER_W, body, 0, unroll=False)

    @pl.when(wid == 0)
    def _():
        col = FULL_BLOCKS * LANE
        pltpu.sync_copy(ttab_hbm.at[:, pl.ds(col, TAIL)], tail_hbm)
        pltpu.sync_copy(tail_hbm, stage64_v)
        for g in range(TAIL // 16):
            rows = lax.iota(jnp.int32, 16) + 16 * g
            for d in range(D_MODEL):
                vals = stage64_v[d, pl.ds(16 * g, 16)]
                plsc.store_scatter(pad_v, [rows, jnp.full((16,), d, jnp.int32)], vals)
        pltpu.sync_copy(pad_v.at[pl.ds(0, TAIL), :], dense_hbm.at[pl.ds(col, TAIL), :])


def _gather_kernel(dense_hbm, idx_hbm, out_hbm, idx_v,
                   r0, r1, r2, r3, g0, g1, g2, g3, s0, s1, s2, s3):
    wid = _wid()
    base = wid * PER_WORKER
    rows = [r0, r1, r2, r3]
    gsem = [g0, g1, g2, g3]
    ssem = [s0, s1, s2, s3]

    pltpu.sync_copy(idx_hbm.at[wid], idx_v)

    def fire_gather(chunk, b):
        pltpu.async_copy(dense_hbm.at[idx_v.at[chunk]], rows[b], gsem[b])

    def wait_gather(chunk, b):
        pltpu.make_async_copy(
            out_hbm.at[pl.ds(base + chunk * LANE, LANE), :], rows[b], gsem[b]
        ).wait()

    def fire_store(chunk, b):
        pltpu.async_copy(
            rows[b], out_hbm.at[pl.ds(base + chunk * LANE, LANE), :], ssem[b]
        )

    def wait_store(chunk, b):
        pltpu.make_async_copy(
            rows[b], out_hbm.at[pl.ds(base + chunk * LANE, LANE), :], ssem[b]
        ).wait()

    def step(chunk, i, refill, fresh):
        wait_gather(chunk, i)
        fire_store(chunk, i)
        bn = (i + 3) % NBUF
        if refill:
            if not fresh:
                wait_store(chunk, bn)
            fire_gather(chunk + 3, bn)

    for b in range(3):
        fire_gather(b, b)

    step(0, 0, True, True)
    step(1, 1, True, False)
    step(2, 2, True, False)
    step(3, 3, True, False)

    def body(t, carry):
        for i in range(NBUF):
            chunk = NBUF * t + i
            wait_gather(chunk, i)
            fire_store(chunk, i)
            bn = (i + 3) % NBUF
            wait_store(chunk, bn)
            fire_gather(chunk + 3, bn)
        return carry

    lax.fori_loop(1, IDX_ROWS // NBUF - 1, body, 0, unroll=False)

    tail = IDX_ROWS - NBUF
    step(tail, 0, True, False)
    step(tail + 1, 1, False, False)
    step(tail + 2, 2, False, False)
    step(tail + 3, 3, False, False)

    for b in range(NBUF):
        wait_store(b, b)


@jax.jit
def _embed(x_blocked, table_t):
    mesh = plsc.VectorSubcoreMesh(core_axis_name="c", subcore_axis_name="s")

    transpose_run = pl.kernel(
        _transpose_kernel,
        out_type=(
            jax.ShapeDtypeStruct((VOCAB_N, PAD_W), jnp.float32),
            jax.ShapeDtypeStruct((TAIL, TAIL), jnp.float32),
        ),
        mesh=mesh,
        scratch_types=[
            pltpu.VMEM((D_MODEL, LANE), jnp.float32),
            pltpu.VMEM((TAIL, TAIL), jnp.float32),
            pltpu.VMEM((LANE, PAD_W), jnp.float32),
        ],
        compiler_params=pltpu.CompilerParams(use_tc_tiling_on_sc=True, needs_layout_passes=False),
    )
    dense, _ = transpose_run(table_t)

    gather_run = pl.kernel(
        _gather_kernel,
        out_type=jax.ShapeDtypeStruct((TOTAL, PAD_W), jnp.float32),
        mesh=mesh,
        scratch_types=[
            pltpu.VMEM((IDX_ROWS, LANE), jnp.int32),
            pltpu.VMEM((LANE, PAD_W), jnp.float32),
            pltpu.VMEM((LANE, PAD_W), jnp.float32),
            pltpu.VMEM((LANE, PAD_W), jnp.float32),
            pltpu.VMEM((LANE, PAD_W), jnp.float32),
            pltpu.SemaphoreType.DMA,
            pltpu.SemaphoreType.DMA,
            pltpu.SemaphoreType.DMA,
            pltpu.SemaphoreType.DMA,
            pltpu.SemaphoreType.DMA,
            pltpu.SemaphoreType.DMA,
            pltpu.SemaphoreType.DMA,
            pltpu.SemaphoreType.DMA,
        ],
        compiler_params=pltpu.CompilerParams(use_tc_tiling_on_sc=True, needs_layout_passes=False),
    )
    out_padded = gather_run(dense, x_blocked)
    return out_padded


def kernel(x, table):
    b, t = x.shape
    x_blocked = x.reshape(NUM_WORKERS, IDX_ROWS, LANE).astype(jnp.int32)
    out_padded = _embed(x_blocked, table.T)
    return out_padded[:, :D_MODEL].reshape(b, t, D_MODEL)


# speedup vs baseline: 1.7776x; 1.7776x over previous
import jax
import jax.numpy as jnp
from jax import lax
from jax.experimental import pallas as pl
from jax.experimental.pallas import tpu as pltpu
from jax.experimental.pallas import tpu_sc as plsc

D_MODEL = 64
PAD_W = 128
NUM_WORKERS = 32
LANE = 128
NBUF = 4
TOTAL = 4096 * 200
PER_WORKER = TOTAL // NUM_WORKERS
IDX_ROWS = PER_WORKER // LANE
VOCAB_N = 1000000
TBLK = 1024


def _transpose_kernel(src_ref, dst_ref):
    t = src_ref[...].T
    dst_ref[...] = jnp.concatenate([t, jnp.zeros_like(t)], axis=1)


def _gather_kernel(dense_hbm, idx_hbm, out_hbm, idx_v,
                   r0, r1, r2, r3, g0, g1, g2, g3, s0, s1, s2, s3):
    wid = lax.axis_index("s") * 2 + lax.axis_index("c")
    base = wid * PER_WORKER
    rows = [r0, r1, r2, r3]
    gsem = [g0, g1, g2, g3]
    ssem = [s0, s1, s2, s3]

    pltpu.sync_copy(idx_hbm.at[wid], idx_v)

    def fire_gather(chunk, b):
        pltpu.async_copy(dense_hbm.at[idx_v.at[chunk]], rows[b], gsem[b])

    def wait_gather(chunk, b):
        pltpu.make_async_copy(
            out_hbm.at[pl.ds(base + chunk * LANE, LANE), :], rows[b], gsem[b]
        ).wait()

    def fire_store(chunk, b):
        pltpu.async_copy(
            rows[b], out_hbm.at[pl.ds(base + chunk * LANE, LANE), :], ssem[b]
        )

    def wait_store(chunk, b):
        pltpu.make_async_copy(
            rows[b], out_hbm.at[pl.ds(base + chunk * LANE, LANE), :], ssem[b]
        ).wait()

    def step(chunk, i, refill, fresh):
        wait_gather(chunk, i)
        fire_store(chunk, i)
        bn = (i + 3) % NBUF
        if refill:
            if not fresh:
                wait_store(chunk, bn)
            fire_gather(chunk + 3, bn)

    for b in range(3):
        fire_gather(b, b)

    step(0, 0, True, True)
    step(1, 1, True, False)
    step(2, 2, True, False)
    step(3, 3, True, False)

    def body(t, carry):
        for i in range(NBUF):
            chunk = NBUF * t + i
            wait_gather(chunk, i)
            fire_store(chunk, i)
            bn = (i + 3) % NBUF
            wait_store(chunk, bn)
            fire_gather(chunk + 3, bn)
        return carry

    lax.fori_loop(1, IDX_ROWS // NBUF - 1, body, 0, unroll=False)

    tail = IDX_ROWS - NBUF
    step(tail, 0, True, False)
    step(tail + 1, 1, False, False)
    step(tail + 2, 2, False, False)
    step(tail + 3, 3, False, False)

    for b in range(NBUF):
        wait_store(b, b)


@jax.jit
def _embed(x_blocked, table_t):
    nblk = (VOCAB_N + TBLK - 1) // TBLK
    dense = pl.pallas_call(
        _transpose_kernel,
        grid=(nblk,),
        in_specs=[pl.BlockSpec((D_MODEL, TBLK), lambda i: (0, i))],
        out_specs=pl.BlockSpec((TBLK, PAD_W), lambda i: (i, 0)),
        out_shape=jax.ShapeDtypeStruct((VOCAB_N, PAD_W), jnp.float32),
    )(table_t)

    mesh = plsc.VectorSubcoreMesh(core_axis_name="c", subcore_axis_name="s")
    gather_run = pl.kernel(
        _gather_kernel,
        out_type=jax.ShapeDtypeStruct((TOTAL, PAD_W), jnp.float32),
        mesh=mesh,
        scratch_types=[
            pltpu.VMEM((IDX_ROWS, LANE), jnp.int32),
            pltpu.VMEM((LANE, PAD_W), jnp.float32),
            pltpu.VMEM((LANE, PAD_W), jnp.float32),
            pltpu.VMEM((LANE, PAD_W), jnp.float32),
            pltpu.VMEM((LANE, PAD_W), jnp.float32),
            pltpu.SemaphoreType.DMA,
            pltpu.SemaphoreType.DMA,
            pltpu.SemaphoreType.DMA,
            pltpu.SemaphoreType.DMA,
            pltpu.SemaphoreType.DMA,
            pltpu.SemaphoreType.DMA,
            pltpu.SemaphoreType.DMA,
            pltpu.SemaphoreType.DMA,
        ],
        compiler_params=pltpu.CompilerParams(use_tc_tiling_on_sc=True, needs_layout_passes=False),
    )
    return gather_run(dense, x_blocked)


def kernel(x, table):
    b, t = x.shape
    x_blocked = x.reshape(NUM_WORKERS, IDX_ROWS, LANE).astype(jnp.int32)
    out_padded = _embed(x_blocked, table.T)
    return out_padded[:, :D_MODEL].reshape(b, t, D_MODEL)


# speedup vs baseline: 2.5764x; 1.4494x over previous
import jax
import jax.numpy as jnp
from jax import lax
from jax.experimental import pallas as pl
from jax.experimental.pallas import tpu as pltpu
from jax.experimental.pallas import tpu_sc as plsc

D_MODEL = 64
PAD_W = 128
NUM_WORKERS = 32
LANE = 128
NBUF = 4
TOTAL = 4096 * 200
PER_WORKER = TOTAL // NUM_WORKERS
IDX_ROWS = PER_WORKER // LANE
VOCAB_N = 1000000
TBLK = 4096


def _transpose_kernel(src_ref, dst_ref):
    t = src_ref[...].T
    dst_ref[...] = jnp.concatenate([t, jnp.zeros_like(t)], axis=1)


def _gather_kernel(dense_hbm, idx_hbm, out_hbm, idx_v,
                   r0, r1, r2, r3, g0, g1, g2, g3, s0, s1, s2, s3):
    wid = lax.axis_index("s") * 2 + lax.axis_index("c")
    base = wid * PER_WORKER
    rows = [r0, r1, r2, r3]
    gsem = [g0, g1, g2, g3]
    ssem = [s0, s1, s2, s3]

    pltpu.sync_copy(idx_hbm.at[wid], idx_v)

    def fire_gather(chunk, b):
        pltpu.async_copy(dense_hbm.at[idx_v.at[chunk]], rows[b], gsem[b])

    def wait_gather(chunk, b):
        pltpu.make_async_copy(
            out_hbm.at[pl.ds(base + chunk * LANE, LANE), :], rows[b], gsem[b]
        ).wait()

    def fire_store(chunk, b):
        pltpu.async_copy(
            rows[b], out_hbm.at[pl.ds(base + chunk * LANE, LANE), :], ssem[b]
        )

    def wait_store(chunk, b):
        pltpu.make_async_copy(
            rows[b], out_hbm.at[pl.ds(base + chunk * LANE, LANE), :], ssem[b]
        ).wait()

    def step(chunk, i, refill, fresh):
        wait_gather(chunk, i)
        fire_store(chunk, i)
        bn = (i + 3) % NBUF
        if refill:
            if not fresh:
                wait_store(chunk, bn)
            fire_gather(chunk + 3, bn)

    for b in range(3):
        fire_gather(b, b)

    step(0, 0, True, True)
    step(1, 1, True, False)
    step(2, 2, True, False)
    step(3, 3, True, False)

    def body(t, carry):
        for i in range(NBUF):
            chunk = NBUF * t + i
            wait_gather(chunk, i)
            fire_store(chunk, i)
            bn = (i + 3) % NBUF
            wait_store(chunk, bn)
            fire_gather(chunk + 3, bn)
        return carry

    lax.fori_loop(1, IDX_ROWS // NBUF - 1, body, 0, unroll=False)

    tail = IDX_ROWS - NBUF
    step(tail, 0, True, False)
    step(tail + 1, 1, False, False)
    step(tail + 2, 2, False, False)
    step(tail + 3, 3, False, False)

    for b in range(NBUF):
        wait_store(b, b)


@jax.jit
def _embed(x_blocked, table_t):
    nblk = (VOCAB_N + TBLK - 1) // TBLK
    dense = pl.pallas_call(
        _transpose_kernel,
        grid=(nblk,),
        in_specs=[pl.BlockSpec((D_MODEL, TBLK), lambda i: (0, i))],
        out_specs=pl.BlockSpec((TBLK, PAD_W), lambda i: (i, 0)),
        out_shape=jax.ShapeDtypeStruct((VOCAB_N, PAD_W), jnp.float32),
    )(table_t)

    mesh = plsc.VectorSubcoreMesh(core_axis_name="c", subcore_axis_name="s")
    gather_run = pl.kernel(
        _gather_kernel,
        out_type=jax.ShapeDtypeStruct((TOTAL, PAD_W), jnp.float32),
        mesh=mesh,
        scratch_types=[
            pltpu.VMEM((IDX_ROWS, LANE), jnp.int32),
            pltpu.VMEM((LANE, PAD_W), jnp.float32),
            pltpu.VMEM((LANE, PAD_W), jnp.float32),
            pltpu.VMEM((LANE, PAD_W), jnp.float32),
            pltpu.VMEM((LANE, PAD_W), jnp.float32),
            pltpu.SemaphoreType.DMA,
            pltpu.SemaphoreType.DMA,
            pltpu.SemaphoreType.DMA,
            pltpu.SemaphoreType.DMA,
            pltpu.SemaphoreType.DMA,
            pltpu.SemaphoreType.DMA,
            pltpu.SemaphoreType.DMA,
            pltpu.SemaphoreType.DMA,
        ],
        compiler_params=pltpu.CompilerParams(use_tc_tiling_on_sc=True, needs_layout_passes=False),
    )
    return gather_run(dense, x_blocked)


def kernel(x, table):
    b, t = x.shape
    x_blocked = x.reshape(NUM_WORKERS, IDX_ROWS, LANE).astype(jnp.int32)
    out_padded = _embed(x_blocked, table.T)
    return out_padded[:, :D_MODEL].reshape(b, t, D_MODEL)


# speedup vs baseline: 2.8271x; 1.0973x over previous
import jax
import jax.numpy as jnp
from jax import lax
from jax.experimental import pallas as pl
from jax.experimental.pallas import tpu as pltpu
from jax.experimental.pallas import tpu_sc as plsc

D_MODEL = 64
PAD_W = 128
NUM_WORKERS = 32
LANE = 128
NBUF = 4
TOTAL = 4096 * 200
PER_WORKER = TOTAL // NUM_WORKERS
IDX_ROWS = PER_WORKER // LANE
VOCAB_N = 1000000
TBLK = 8192


def _transpose_kernel(src_ref, dst_ref):
    t = src_ref[...].T
    dst_ref[...] = jnp.concatenate([t, jnp.zeros_like(t)], axis=1)


def _gather_kernel(dense_hbm, idx_hbm, out_hbm, idx_v,
                   r0, r1, r2, r3, g0, g1, g2, g3, s0, s1, s2, s3):
    wid = lax.axis_index("s") * 2 + lax.axis_index("c")
    base = wid * PER_WORKER
    rows = [r0, r1, r2, r3]
    gsem = [g0, g1, g2, g3]
    ssem = [s0, s1, s2, s3]

    pltpu.sync_copy(idx_hbm.at[wid], idx_v)

    def fire_gather(chunk, b):
        pltpu.async_copy(dense_hbm.at[idx_v.at[chunk]], rows[b], gsem[b])

    def wait_gather(chunk, b):
        pltpu.make_async_copy(
            out_hbm.at[pl.ds(base + chunk * LANE, LANE), :], rows[b], gsem[b]
        ).wait()

    def fire_store(chunk, b):
        pltpu.async_copy(
            rows[b], out_hbm.at[pl.ds(base + chunk * LANE, LANE), :], ssem[b]
        )

    def wait_store(chunk, b):
        pltpu.make_async_copy(
            rows[b], out_hbm.at[pl.ds(base + chunk * LANE, LANE), :], ssem[b]
        ).wait()

    def step(chunk, i, refill, fresh):
        wait_gather(chunk, i)
        fire_store(chunk, i)
        bn = (i + 3) % NBUF
        if refill:
            if not fresh:
                wait_store(chunk, bn)
            fire_gather(chunk + 3, bn)

    for b in range(3):
        fire_gather(b, b)

    step(0, 0, True, True)
    step(1, 1, True, False)
    step(2, 2, True, False)
    step(3, 3, True, False)

    def body(t, carry):
        for i in range(NBUF):
            chunk = NBUF * t + i
            wait_gather(chunk, i)
            fire_store(chunk, i)
            bn = (i + 3) % NBUF
            wait_store(chunk, bn)
            fire_gather(chunk + 3, bn)
        return carry

    lax.fori_loop(1, IDX_ROWS // NBUF - 1, body, 0, unroll=False)

    tail = IDX_ROWS - NBUF
    step(tail, 0, True, False)
    step(tail + 1, 1, False, False)
    step(tail + 2, 2, False, False)
    step(tail + 3, 3, False, False)

    for b in range(NBUF):
        wait_store(b, b)


@jax.jit
def _embed(x_blocked, table_t):
    nblk = (VOCAB_N + TBLK - 1) // TBLK
    dense = pl.pallas_call(
        _transpose_kernel,
        grid=(nblk,),
        in_specs=[pl.BlockSpec((D_MODEL, TBLK), lambda i: (0, i))],
        out_specs=pl.BlockSpec((TBLK, PAD_W), lambda i: (i, 0)),
        out_shape=jax.ShapeDtypeStruct((VOCAB_N, PAD_W), jnp.float32),
    )(table_t)

    mesh = plsc.VectorSubcoreMesh(core_axis_name="c", subcore_axis_name="s")
    gather_run = pl.kernel(
        _gather_kernel,
        out_type=jax.ShapeDtypeStruct((TOTAL, PAD_W), jnp.float32),
        mesh=mesh,
        scratch_types=[
            pltpu.VMEM((IDX_ROWS, LANE), jnp.int32),
            pltpu.VMEM((LANE, PAD_W), jnp.float32),
            pltpu.VMEM((LANE, PAD_W), jnp.float32),
            pltpu.VMEM((LANE, PAD_W), jnp.float32),
            pltpu.VMEM((LANE, PAD_W), jnp.float32),
            pltpu.SemaphoreType.DMA,
            pltpu.SemaphoreType.DMA,
            pltpu.SemaphoreType.DMA,
            pltpu.SemaphoreType.DMA,
            pltpu.SemaphoreType.DMA,
            pltpu.SemaphoreType.DMA,
            pltpu.SemaphoreType.DMA,
            pltpu.SemaphoreType.DMA,
        ],
        compiler_params=pltpu.CompilerParams(use_tc_tiling_on_sc=True, needs_layout_passes=False),
    )
    return gather_run(dense, x_blocked)


def kernel(x, table):
    b, t = x.shape
    x_blocked = x.reshape(NUM_WORKERS, IDX_ROWS, LANE).astype(jnp.int32)
    out_padded = _embed(x_blocked, table.T)
    return out_padded[:, :D_MODEL].reshape(b, t, D_MODEL)


# speedup vs baseline: 2.9004x; 1.0259x over previous
import jax
import jax.numpy as jnp
from jax import lax
from jax.experimental import pallas as pl
from jax.experimental.pallas import tpu as pltpu
from jax.experimental.pallas import tpu_sc as plsc

D_MODEL = 64
PAD_W = 128
NUM_WORKERS = 32
LANE = 128
NBUF = 4
TOTAL = 4096 * 200
PER_WORKER = TOTAL // NUM_WORKERS
IDX_ROWS = PER_WORKER // LANE
VOCAB_N = 1000000
TBLK = 16384


def _transpose_kernel(src_ref, dst_ref):
    t = src_ref[...].T
    dst_ref[...] = jnp.concatenate([t, jnp.zeros_like(t)], axis=1)


def _gather_kernel(dense_hbm, idx_hbm, out_hbm, idx_v,
                   r0, r1, r2, r3, g0, g1, g2, g3, s0, s1, s2, s3):
    wid = lax.axis_index("s") * 2 + lax.axis_index("c")
    base = wid * PER_WORKER
    rows = [r0, r1, r2, r3]
    gsem = [g0, g1, g2, g3]
    ssem = [s0, s1, s2, s3]

    pltpu.sync_copy(idx_hbm.at[wid], idx_v)

    def fire_gather(chunk, b):
        pltpu.async_copy(dense_hbm.at[idx_v.at[chunk]], rows[b], gsem[b])

    def wait_gather(chunk, b):
        pltpu.make_async_copy(
            out_hbm.at[pl.ds(base + chunk * LANE, LANE), :], rows[b], gsem[b]
        ).wait()

    def fire_store(chunk, b):
        pltpu.async_copy(
            rows[b], out_hbm.at[pl.ds(base + chunk * LANE, LANE), :], ssem[b]
        )

    def wait_store(chunk, b):
        pltpu.make_async_copy(
            rows[b], out_hbm.at[pl.ds(base + chunk * LANE, LANE), :], ssem[b]
        ).wait()

    def step(chunk, i, refill, fresh):
        wait_gather(chunk, i)
        fire_store(chunk, i)
        bn = (i + 3) % NBUF
        if refill:
            if not fresh:
                wait_store(chunk, bn)
            fire_gather(chunk + 3, bn)

    for b in range(3):
        fire_gather(b, b)

    step(0, 0, True, True)
    step(1, 1, True, False)
    step(2, 2, True, False)
    step(3, 3, True, False)

    def body(t, carry):
        for i in range(NBUF):
            chunk = NBUF * t + i
            wait_gather(chunk, i)
            fire_store(chunk, i)
            bn = (i + 3) % NBUF
            wait_store(chunk, bn)
            fire_gather(chunk + 3, bn)
        return carry

    lax.fori_loop(1, IDX_ROWS // NBUF - 1, body, 0, unroll=False)

    tail = IDX_ROWS - NBUF
    step(tail, 0, True, False)
    step(tail + 1, 1, False, False)
    step(tail + 2, 2, False, False)
    step(tail + 3, 3, False, False)

    for b in range(NBUF):
        wait_store(b, b)


@jax.jit
def _embed(x_blocked, table_t):
    nblk = (VOCAB_N + TBLK - 1) // TBLK
    dense = pl.pallas_call(
        _transpose_kernel,
        grid=(nblk,),
        in_specs=[pl.BlockSpec((D_MODEL, TBLK), lambda i: (0, i))],
        out_specs=pl.BlockSpec((TBLK, PAD_W), lambda i: (i, 0)),
        out_shape=jax.ShapeDtypeStruct((VOCAB_N, PAD_W), jnp.float32),
    )(table_t)

    mesh = plsc.VectorSubcoreMesh(core_axis_name="c", subcore_axis_name="s")
    gather_run = pl.kernel(
        _gather_kernel,
        out_type=jax.ShapeDtypeStruct((TOTAL, PAD_W), jnp.float32),
        mesh=mesh,
        scratch_types=[
            pltpu.VMEM((IDX_ROWS, LANE), jnp.int32),
            pltpu.VMEM((LANE, PAD_W), jnp.float32),
            pltpu.VMEM((LANE, PAD_W), jnp.float32),
            pltpu.VMEM((LANE, PAD_W), jnp.float32),
            pltpu.VMEM((LANE, PAD_W), jnp.float32),
            pltpu.SemaphoreType.DMA,
            pltpu.SemaphoreType.DMA,
            pltpu.SemaphoreType.DMA,
            pltpu.SemaphoreType.DMA,
            pltpu.SemaphoreType.DMA,
            pltpu.SemaphoreType.DMA,
            pltpu.SemaphoreType.DMA,
            pltpu.SemaphoreType.DMA,
        ],
        compiler_params=pltpu.CompilerParams(use_tc_tiling_on_sc=True, needs_layout_passes=False),
    )
    return gather_run(dense, x_blocked)


def kernel(x, table):
    b, t = x.shape
    x_blocked = x.reshape(NUM_WORKERS, IDX_ROWS, LANE).astype(jnp.int32)
    out_padded = _embed(x_blocked, table.T)
    return out_padded[:, :D_MODEL].reshape(b, t, D_MODEL)


# speedup vs baseline: 2.9302x; 1.0103x over previous
import jax
import jax.numpy as jnp
from jax import lax
from jax.experimental import pallas as pl
from jax.experimental.pallas import tpu as pltpu
from jax.experimental.pallas import tpu_sc as plsc

D_MODEL = 64
PAD_W = 128
NUM_WORKERS = 32
LANE = 128
NBUF = 4
TOTAL = 4096 * 200
PER_WORKER = TOTAL // NUM_WORKERS
IDX_ROWS = PER_WORKER // LANE
VOCAB_N = 1000000
TBLK = 32768


def _transpose_kernel(src_ref, dst_ref):
    t = src_ref[...].T
    dst_ref[...] = jnp.concatenate([t, jnp.zeros_like(t)], axis=1)


def _gather_kernel(dense_hbm, idx_hbm, out_hbm, idx_v,
                   r0, r1, r2, r3, g0, g1, g2, g3, s0, s1, s2, s3):
    wid = lax.axis_index("s") * 2 + lax.axis_index("c")
    base = wid * PER_WORKER
    rows = [r0, r1, r2, r3]
    gsem = [g0, g1, g2, g3]
    ssem = [s0, s1, s2, s3]

    pltpu.sync_copy(idx_hbm.at[wid], idx_v)

    def fire_gather(chunk, b):
        pltpu.async_copy(dense_hbm.at[idx_v.at[chunk]], rows[b], gsem[b])

    def wait_gather(chunk, b):
        pltpu.make_async_copy(
            out_hbm.at[pl.ds(base + chunk * LANE, LANE), :], rows[b], gsem[b]
        ).wait()

    def fire_store(chunk, b):
        pltpu.async_copy(
            rows[b], out_hbm.at[pl.ds(base + chunk * LANE, LANE), :], ssem[b]
        )

    def wait_store(chunk, b):
        pltpu.make_async_copy(
            rows[b], out_hbm.at[pl.ds(base + chunk * LANE, LANE), :], ssem[b]
        ).wait()

    def step(chunk, i, refill, fresh):
        wait_gather(chunk, i)
        fire_store(chunk, i)
        bn = (i + 3) % NBUF
        if refill:
            if not fresh:
                wait_store(chunk, bn)
            fire_gather(chunk + 3, bn)

    for b in range(3):
        fire_gather(b, b)

    step(0, 0, True, True)
    step(1, 1, True, False)
    step(2, 2, True, False)
    step(3, 3, True, False)

    def body(t, carry):
        for i in range(NBUF):
            chunk = NBUF * t + i
            wait_gather(chunk, i)
            fire_store(chunk, i)
            bn = (i + 3) % NBUF
            wait_store(chunk, bn)
            fire_gather(chunk + 3, bn)
        return carry

    lax.fori_loop(1, IDX_ROWS // NBUF - 1, body, 0, unroll=False)

    tail = IDX_ROWS - NBUF
    step(tail, 0, True, False)
    step(tail + 1, 1, False, False)
    step(tail + 2, 2, False, False)
    step(tail + 3, 3, False, False)

    for b in range(NBUF):
        wait_store(b, b)


@jax.jit
def _embed(x_blocked, table_t):
    nblk = (VOCAB_N + TBLK - 1) // TBLK
    dense = pl.pallas_call(
        _transpose_kernel,
        grid=(nblk,),
        in_specs=[pl.BlockSpec((D_MODEL, TBLK), lambda i: (0, i))],
        out_specs=pl.BlockSpec((TBLK, PAD_W), lambda i: (i, 0)),
        out_shape=jax.ShapeDtypeStruct((VOCAB_N, PAD_W), jnp.float32),
    )(table_t)

    mesh = plsc.VectorSubcoreMesh(core_axis_name="c", subcore_axis_name="s")
    gather_run = pl.kernel(
        _gather_kernel,
        out_type=jax.ShapeDtypeStruct((TOTAL, PAD_W), jnp.float32),
        mesh=mesh,
        scratch_types=[
            pltpu.VMEM((IDX_ROWS, LANE), jnp.int32),
            pltpu.VMEM((LANE, PAD_W), jnp.float32),
            pltpu.VMEM((LANE, PAD_W), jnp.float32),
            pltpu.VMEM((LANE, PAD_W), jnp.float32),
            pltpu.VMEM((LANE, PAD_W), jnp.float32),
            pltpu.SemaphoreType.DMA,
            pltpu.SemaphoreType.DMA,
            pltpu.SemaphoreType.DMA,
            pltpu.SemaphoreType.DMA,
            pltpu.SemaphoreType.DMA,
            pltpu.SemaphoreType.DMA,
            pltpu.SemaphoreType.DMA,
            pltpu.SemaphoreType.DMA,
        ],
        compiler_params=pltpu.CompilerParams(use_tc_tiling_on_sc=True, needs_layout_passes=False),
    )
    return gather_run(dense, x_blocked)


def kernel(x, table):
    b, t = x.shape
    x_blocked = x.reshape(NUM_WORKERS, IDX_ROWS, LANE).astype(jnp.int32)
    out_padded = _embed(x_blocked, table.T)
    return out_padded[:, :D_MODEL].reshape(b, t, D_MODEL)
